# Initial kernel scaffold; baseline (speedup 1.0000x reference)
import jax
import jax.numpy as jnp
from jax.experimental import pallas as pl


def kernel(x, edge_index, W1, b1, W2, b2):
    raise NotImplementedError("write your pallas kernel here")


# speedup vs baseline: 13.8447x; 13.8447x over previous
import functools

import jax
import jax.numpy as jnp
from jax import lax
from jax.experimental import pallas as pl
from jax.experimental.pallas import tpu as pltpu
from jax.experimental.pallas import tpu_sc as plsc

N = 10000
E = 320000
F_IN = 128
HID = 64
OUT = 32

NC = 2
NS = 16
NW = NC * NS
N_PAD = 10240
EPT = E // NW
ECH = 10000
NCHUNK = E // ECH


def _zero_vmem_1d(ref, n):
    def body(i, _):
        ref[pl.ds(pl.multiple_of(i * 16, 16), 16)] = jnp.zeros(
            (16,), jnp.float32)
        return ()
    lax.fori_loop(0, n // 16, body, ())


def _make_deg_kernel():
    mesh = plsc.VectorSubcoreMesh(core_axis_name="c", subcore_axis_name="s")

    @functools.partial(
        pl.kernel,
        out_type=jax.ShapeDtypeStruct((NW, N_PAD), jnp.float32),
        mesh=mesh,
        compiler_params=pltpu.CompilerParams(
            needs_layout_passes=False, use_tc_tiling_on_sc=False),
        scratch_types=[
            pltpu.VMEM((N_PAD,), jnp.float32),
            pltpu.VMEM((EPT,), jnp.int32),
        ],
    )
    def deg_kernel(dst_hbm, out_hbm, hist_v, ed_v):
        c = lax.axis_index("c")
        s = lax.axis_index("s")
        wid = c * NS + s
        _zero_vmem_1d(hist_v, N_PAD)
        pltpu.sync_copy(dst_hbm.at[pl.ds(wid * EPT, EPT)], ed_v)
        ones16 = jnp.full((16,), 1.0, jnp.float32)

        def body(g, _):
            d16 = ed_v[pl.ds(pl.multiple_of(g * 16, 16), 16)]
            plsc.addupdate_scatter(hist_v, [d16], ones16)
            return ()

        lax.fori_loop(0, EPT // 16, body, ())
        pltpu.sync_copy(hist_v, out_hbm.at[wid])

    return deg_kernel


def _make_prop_kernel(FPT):
    mesh = plsc.VectorSubcoreMesh(core_axis_name="c", subcore_axis_name="s")
    CSZ = FPT * N_PAD

    @functools.partial(
        pl.kernel,
        out_type=jax.ShapeDtypeStruct((NW * CSZ,), jnp.float32),
        mesh=mesh,
        compiler_params=pltpu.CompilerParams(
            needs_layout_passes=False, use_tc_tiling_on_sc=False),
        scratch_types=[
            pltpu.VMEM((CSZ,), jnp.float32),
            pltpu.VMEM((CSZ,), jnp.float32),
            pltpu.VMEM((ECH,), jnp.int32),
            pltpu.VMEM((ECH,), jnp.int32),
        ],
    )
    def prop_kernel(g_hbm, src_hbm, dst_hbm, out_hbm, gcols, acc, es_v, ed_v):
        c = lax.axis_index("c")
        s = lax.axis_index("s")
        wid = c * NS + s
        pltpu.sync_copy(g_hbm.at[pl.ds(wid * CSZ, CSZ)], gcols)
        _zero_vmem_1d(acc, CSZ)

        def group_body(g, _):
            base = pl.ds(pl.multiple_of(g * 16, 16), 16)
            s16 = es_v[base]
            d16 = ed_v[base]
            for f in range(FPT):
                off = f * N_PAD
                v = plsc.load_gather(gcols, [s16 + off])
                plsc.addupdate_scatter(acc, [d16 + off], v)
            return ()

        def chunk_body(ch, _):
            eoff = ch * ECH
            pltpu.sync_copy(src_hbm.at[pl.ds(eoff, ECH)], es_v)
            pltpu.sync_copy(dst_hbm.at[pl.ds(eoff, ECH)], ed_v)
            lax.fori_loop(0, ECH // 16, group_body, ())
            return ()

        lax.fori_loop(0, NCHUNK, chunk_body, ())
        pltpu.sync_copy(acc, out_hbm.at[pl.ds(wid * CSZ, CSZ)])

    return prop_kernel


def _tc1_body(xp_ref, w1_ref, degp_ref, g1t_ref, dinv_ref):
    deg = 1.0 + jnp.sum(degp_ref[...], axis=0, keepdims=True)
    dinv = lax.rsqrt(deg)
    xw_t = lax.dot_general(w1_ref[...], xp_ref[...],
                           (((0,), (1,)), ((), ())),
                           preferred_element_type=jnp.float32)
    g1t_ref[...] = xw_t * dinv
    dinv_ref[...] = dinv


def _tc2_body(p1_ref, g1t_ref, dinv_ref, w2_ref, b1_ref, g2t_ref):
    h1 = jnp.maximum(
        (p1_ref[...] + g1t_ref[...]) * dinv_ref[...] + b1_ref[...], 0.0)
    g2t_ref[...] = lax.dot_general(w2_ref[...], h1,
                                   (((0,), (0,)), ((), ())),
                                   preferred_element_type=jnp.float32
                                   ) * dinv_ref[...]


def _tc3_body(p2_ref, g2t_ref, dinv_ref, b2_ref, out_ref):
    h2 = jnp.maximum(
        (p2_ref[:, :N] + g2t_ref[:, :N]) * dinv_ref[:, :N] + b2_ref[...],
        0.0)
    out_ref[...] = jnp.sum(h2, axis=1, keepdims=True) * (1.0 / N)


_deg_kernel = _make_deg_kernel()
_prop_hid = _make_prop_kernel(HID // NW)
_prop_out = _make_prop_kernel(OUT // NW)


@jax.jit
def kernel(x, edge_index, W1, b1, W2, b2):
    src = edge_index[0]
    dst = edge_index[1]
    x_pad = jnp.zeros((N_PAD, F_IN), jnp.float32).at[:N].set(x)

    degp = _deg_kernel(dst)

    g1t, dinv = pl.pallas_call(
        _tc1_body,
        out_shape=[
            jax.ShapeDtypeStruct((HID, N_PAD), jnp.float32),
            jax.ShapeDtypeStruct((1, N_PAD), jnp.float32),
        ],
    )(x_pad, W1, degp)

    p1 = _prop_hid(g1t.reshape(-1), src, dst)

    g2t = pl.pallas_call(
        _tc2_body,
        out_shape=jax.ShapeDtypeStruct((OUT, N_PAD), jnp.float32),
    )(p1.reshape(HID, N_PAD), g1t, dinv, W2, b1.reshape(HID, 1))

    p2 = _prop_out(g2t.reshape(-1), src, dst)

    pooled = pl.pallas_call(
        _tc3_body,
        out_shape=jax.ShapeDtypeStruct((OUT, 1), jnp.float32),
    )(p2.reshape(OUT, N_PAD), g2t, dinv, b2.reshape(OUT, 1))
    return pooled.reshape(1, OUT)
